# baseline (speedup 1.0000x reference)
import dataclasses
import functools

import jax
import jax.numpy as jnp
from jax import lax
from jax.experimental import pallas as pl
from jax.experimental.pallas import tpu as pltpu
from jax.experimental.pallas import tpu_sc as plsc

_NC = 2
_NS = 16
_NW = _NC * _NS
_L = 16


def _sc_compiler_params():
    cp = pltpu.CompilerParams()
    if "needs_layout_passes" in pltpu.CompilerParams.__dataclass_fields__:
        cp = dataclasses.replace(cp, needs_layout_passes=False)
    return cp


def _xwT_call(x, W, n, d_in, d_hid):
    def body(x_ref, w_ref, o_ref):
        xw = jnp.dot(x_ref[...], w_ref[...], preferred_element_type=jnp.float32)
        o_ref[...] = xw.T

    return pl.pallas_call(
        body,
        out_shape=jax.ShapeDtypeStruct((d_hid, n), jnp.float32),
    )(x, W)


def _edge_chunk(e):
    tiles128 = e // 128
    base_t = tiles128 // _NW
    rem = tiles128 - base_t * _NW
    return base_t * 128, rem


def _sc_degree_call(edges2, n, e):
    base_len, rem = _edge_chunk(e)
    big_len = base_len + 128
    mesh = plsc.VectorSubcoreMesh(
        core_axis_name="c", subcore_axis_name="s",
        num_cores=_NC, num_subcores=_NS)

    n_pad = _pad_up(n, 1024)

    @functools.partial(
        pl.kernel,
        out_type=jax.ShapeDtypeStruct((_NW, n_pad), jnp.float32),
        mesh=mesh,
        scratch_types=[
            pltpu.VMEM((2, big_len), jnp.int32),
            pltpu.VMEM((n_pad,), jnp.float32),
            pltpu.SemaphoreType.DMA,
        ],
        compiler_params=_sc_compiler_params(),
    )
    def hist_kernel(edges_hbm, out_hbm, slab_v, hist_v, sem):
        cid = lax.axis_index("c")
        sid = lax.axis_index("s")
        wid = cid * _NS + sid
        base = wid * base_len + jnp.minimum(wid, rem) * 128
        zeros = jnp.zeros((_L,), jnp.float32)
        ones = jnp.ones((_L,), jnp.float32)

        def work(clen):
            cp = pltpu.async_copy(
                edges_hbm.at[:, pl.ds(base, clen)],
                slab_v.at[:, pl.ds(0, clen)], sem)

            @plsc.parallel_loop(0, n_pad, step=_L, unroll=8)
            def _(i):
                hist_v[pl.ds(i, _L)] = zeros

            cp.wait()

            @plsc.parallel_loop(0, clen, step=_L, unroll=8)
            def _(i):
                d = slab_v[1, pl.ds(i, _L)]
                plsc.addupdate_scatter(hist_v, [d], ones)

        @pl.when(wid < rem)
        def _():
            work(big_len)

        @pl.when(wid >= rem)
        def _():
            work(base_len)

        pltpu.sync_copy(hist_v, out_hbm.at[wid])

    return hist_kernel(edges2)


def _pad_up(v, m):
    return ((v + m - 1) // m) * m


def _dinv_y_call(deg_part, xwT, n, d_hid):
    def body(dp_ref, xwT_ref, y_ref, yflat_ref, dinv_ref):
        deg = jnp.sum(dp_ref[:, :n], axis=0) + 1.0
        dinv = lax.rsqrt(deg)
        dinv_ref[...] = dinv[None, :]
        y = xwT_ref[...] * dinv[None, :]
        y_ref[...] = y
        for c in range(d_hid):
            yflat_ref[pl.ds(c * n, n)] = y[c]

    return pl.pallas_call(
        body,
        out_shape=[
            jax.ShapeDtypeStruct((d_hid, n), jnp.float32),
            jax.ShapeDtypeStruct((d_hid * n,), jnp.float32),
            jax.ShapeDtypeStruct((1, n), jnp.float32),
        ],
    )(deg_part, xwT)


def _sc_msgpass_call(edges2, y_flat, n, e, d_hid):
    base_len, rem = _edge_chunk(e)
    big_len = base_len + 128
    fn = d_hid * n
    p_stride = _pad_up(n, 1024)
    fn_pad = d_hid * p_stride
    mesh = plsc.VectorSubcoreMesh(
        core_axis_name="c", subcore_axis_name="s",
        num_cores=_NC, num_subcores=_NS)

    @functools.partial(
        pl.kernel,
        out_type=jax.ShapeDtypeStruct((_NW, fn_pad), jnp.float32),
        mesh=mesh,
        scratch_types=[
            pltpu.VMEM((2, big_len), jnp.int32),
            pltpu.VMEM((fn,), jnp.float32),
            pltpu.VMEM((fn_pad,), jnp.float32),
            pltpu.SemaphoreType.DMA,
        ],
        compiler_params=_sc_compiler_params(),
    )
    def msg_kernel(edges_hbm, y_hbm, out_hbm,
                   slab_v, y_v, agg_v, sem):
        cid = lax.axis_index("c")
        sid = lax.axis_index("s")
        wid = cid * _NS + sid
        base = wid * base_len + jnp.minimum(wid, rem) * 128
        zeros = jnp.zeros((_L,), jnp.float32)
        nvec = jnp.full((_L,), n, jnp.int32)
        pvec = jnp.full((_L,), p_stride, jnp.int32)

        def work(clen):
            cp1 = pltpu.async_copy(
                edges_hbm.at[:, pl.ds(base, clen)],
                slab_v.at[:, pl.ds(0, clen)], sem)
            cp3 = pltpu.async_copy(y_hbm, y_v, sem)

            @plsc.parallel_loop(0, fn_pad, step=_L, unroll=8)
            def _(i):
                agg_v[pl.ds(i, _L)] = zeros

            cp1.wait()
            cp3.wait()

            @plsc.parallel_loop(0, clen, step=_L, unroll=16)
            def _(i):
                s0 = slab_v[0, pl.ds(i, _L)]
                d0 = slab_v[1, pl.ds(i, _L)]
                s1 = s0 + nvec
                s2 = s1 + nvec
                d1 = d0 + pvec
                d2 = d1 + pvec
                v0 = plsc.load_gather(y_v, [s0])
                v1 = plsc.load_gather(y_v, [s1])
                v2 = plsc.load_gather(y_v, [s2])
                plsc.addupdate_scatter(agg_v, [d0], v0)
                plsc.addupdate_scatter(agg_v, [d1], v1)
                plsc.addupdate_scatter(agg_v, [d2], v2)

        @pl.when(wid < rem)
        def _():
            work(big_len)

        @pl.when(wid >= rem)
        def _():
            work(base_len)

        pltpu.sync_copy(agg_v, out_hbm.at[wid])

    return msg_kernel(edges2, y_flat)


def _final_call(agg_part, y, dinv, b, W_lin, b_lin, n, p_stride, d_hid,
                d_out):
    def body(ap_ref, y_ref, dinv_ref, b_ref, wl_ref, bl_ref, h_ref, z_ref):
        accr = jnp.sum(ap_ref[...], axis=0, keepdims=True)
        aggs = jnp.concatenate(
            [accr[:, c * p_stride:c * p_stride + n]
             for c in range(d_hid)], axis=0)
        dinv = dinv_ref[...]
        aggc = (aggs + y_ref[...]) * dinv
        b_col = b_ref[...].reshape(d_hid, 1)
        h_cols = jnp.maximum(aggc + b_col, 0.0)
        z_cols = lax.dot_general(
            wl_ref[...], h_cols, (((0,), (0,)), ((), ())),
            preferred_element_type=jnp.float32)
        z_cols = z_cols + bl_ref[...].reshape(d_out, 1)
        h_ref[...] = h_cols
        z_ref[...] = z_cols

    return pl.pallas_call(
        body,
        out_shape=[
            jax.ShapeDtypeStruct((d_hid, n), jnp.float32),
            jax.ShapeDtypeStruct((d_out, n), jnp.float32),
        ],
    )(agg_part, y, dinv, b, W_lin, b_lin)


def kernel(x, edges, W, b, W_lin, b_lin):
    n, d_in = x.shape
    d_hid = W.shape[1]
    d_out = W_lin.shape[1]
    e = edges.shape[1]
    assert e % (_NW * _L) == 0 and n % _L == 0

    edges = edges.astype(jnp.int32)

    xwT = _xwT_call(x, W, n, d_in, d_hid)
    deg_part = _sc_degree_call(edges, n, e)
    y, y_flat, dinv = _dinv_y_call(deg_part, xwT, n, d_hid)
    agg_part = _sc_msgpass_call(edges, y_flat, n, e, d_hid)
    p_stride = agg_part.shape[1] // d_hid
    h_cols, z_cols = _final_call(agg_part, y, dinv, b, W_lin,
                                 b_lin, n, p_stride, d_hid, d_out)
    return (h_cols.T, z_cols.T)


# speedup vs baseline: 1.0035x; 1.0035x over previous
import dataclasses
import functools

import jax
import jax.numpy as jnp
from jax import lax
from jax.experimental import pallas as pl
from jax.experimental.pallas import tpu as pltpu
from jax.experimental.pallas import tpu_sc as plsc

_NC = 2
_NS = 16
_NW = _NC * _NS
_L = 16


def _sc_compiler_params():
    cp = pltpu.CompilerParams()
    if "needs_layout_passes" in pltpu.CompilerParams.__dataclass_fields__:
        cp = dataclasses.replace(cp, needs_layout_passes=False)
    return cp


def _xwT_call(x, W, n, d_in, d_hid):
    def body(x_ref, w_ref, o_ref):
        xw = jnp.dot(x_ref[...], w_ref[...], preferred_element_type=jnp.float32)
        o_ref[...] = xw.T

    return pl.pallas_call(
        body,
        out_shape=jax.ShapeDtypeStruct((d_hid, n), jnp.float32),
    )(x, W)


def _edge_chunk(e):
    tiles128 = e // 128
    base_t = tiles128 // _NW
    rem = tiles128 - base_t * _NW
    return base_t * 128, rem


def _sc_degree_call(edges2, n, e):
    base_len, rem = _edge_chunk(e)
    big_len = base_len + 128
    mesh = plsc.VectorSubcoreMesh(
        core_axis_name="c", subcore_axis_name="s",
        num_cores=_NC, num_subcores=_NS)

    n_pad = _pad_up(n, 1024)

    @functools.partial(
        pl.kernel,
        out_type=jax.ShapeDtypeStruct((_NW, n_pad), jnp.float32),
        mesh=mesh,
        scratch_types=[
            pltpu.VMEM((2, big_len), jnp.int32),
            pltpu.VMEM((n_pad,), jnp.float32),
            pltpu.SemaphoreType.DMA,
        ],
        compiler_params=_sc_compiler_params(),
    )
    def hist_kernel(edges_hbm, out_hbm, slab_v, hist_v, sem):
        cid = lax.axis_index("c")
        sid = lax.axis_index("s")
        wid = cid * _NS + sid
        base = wid * base_len + jnp.minimum(wid, rem) * 128
        zeros = jnp.zeros((_L,), jnp.float32)
        ones = jnp.ones((_L,), jnp.float32)

        def work(clen):
            cp = pltpu.async_copy(
                edges_hbm.at[:, pl.ds(base, clen)],
                slab_v.at[:, pl.ds(0, clen)], sem)

            @plsc.parallel_loop(0, n_pad, step=_L, unroll=8)
            def _(i):
                hist_v[pl.ds(i, _L)] = zeros

            cp.wait()

            @plsc.parallel_loop(0, clen, step=_L, unroll=8)
            def _(i):
                d = slab_v[1, pl.ds(i, _L)]
                plsc.addupdate_scatter(hist_v, [d], ones)

        @pl.when(wid < rem)
        def _():
            work(big_len)

        @pl.when(wid >= rem)
        def _():
            work(base_len)

        pltpu.sync_copy(hist_v, out_hbm.at[wid])

    return hist_kernel(edges2)


def _pad_up(v, m):
    return ((v + m - 1) // m) * m


def _dinv_y_call(deg_part, xwT, n, d_hid):
    def body(dp_ref, xwT_ref, y_ref, yflat_ref, dinv_ref):
        deg = jnp.sum(dp_ref[:, :n], axis=0) + 1.0
        dinv = lax.rsqrt(deg)
        dinv_ref[...] = dinv[None, :]
        y = xwT_ref[...] * dinv[None, :]
        y_ref[...] = y
        for c in range(d_hid):
            yflat_ref[pl.ds(c * n, n)] = y[c]

    return pl.pallas_call(
        body,
        out_shape=[
            jax.ShapeDtypeStruct((d_hid, n), jnp.float32),
            jax.ShapeDtypeStruct((d_hid * n,), jnp.float32),
            jax.ShapeDtypeStruct((1, n), jnp.float32),
        ],
    )(deg_part, xwT)


def _sc_msgpass_call(edges2, y_flat, n, e, d_hid):
    base_len, rem = _edge_chunk(e)
    big_len = base_len + 128
    fn = d_hid * n
    p_stride = _pad_up(n, 1024)
    fn_pad = d_hid * p_stride
    mesh = plsc.VectorSubcoreMesh(
        core_axis_name="c", subcore_axis_name="s",
        num_cores=_NC, num_subcores=_NS)

    @functools.partial(
        pl.kernel,
        out_type=jax.ShapeDtypeStruct((_NW, fn_pad), jnp.float32),
        mesh=mesh,
        scratch_types=[
            pltpu.VMEM((2, big_len), jnp.int32),
            pltpu.VMEM((fn,), jnp.float32),
            pltpu.VMEM((fn_pad,), jnp.float32),
            pltpu.SemaphoreType.DMA,
        ],
        compiler_params=_sc_compiler_params(),
    )
    def msg_kernel(edges_hbm, y_hbm, out_hbm,
                   slab_v, y_v, agg_v, sem):
        cid = lax.axis_index("c")
        sid = lax.axis_index("s")
        wid = cid * _NS + sid
        base = wid * base_len + jnp.minimum(wid, rem) * 128
        zeros = jnp.zeros((_L,), jnp.float32)
        nvec = jnp.full((_L,), n, jnp.int32)
        pvec = jnp.full((_L,), p_stride, jnp.int32)

        def work(clen):
            cp1 = pltpu.async_copy(
                edges_hbm.at[:, pl.ds(base, clen)],
                slab_v.at[:, pl.ds(0, clen)], sem)
            cp3 = pltpu.async_copy(y_hbm, y_v, sem)

            @plsc.parallel_loop(0, fn_pad, step=_L, unroll=8)
            def _(i):
                agg_v[pl.ds(i, _L)] = zeros

            cp1.wait()
            cp3.wait()

            @plsc.parallel_loop(0, clen, step=_L, unroll=8)
            def _(i):
                s0 = slab_v[0, pl.ds(i, _L)]
                d0 = slab_v[1, pl.ds(i, _L)]
                s1 = s0 + nvec
                s2 = s1 + nvec
                d1 = d0 + pvec
                d2 = d1 + pvec
                v0 = plsc.load_gather(y_v, [s0])
                v1 = plsc.load_gather(y_v, [s1])
                v2 = plsc.load_gather(y_v, [s2])
                plsc.addupdate_scatter(agg_v, [d0], v0)
                plsc.addupdate_scatter(agg_v, [d1], v1)
                plsc.addupdate_scatter(agg_v, [d2], v2)

        @pl.when(wid < rem)
        def _():
            work(big_len)

        @pl.when(wid >= rem)
        def _():
            work(base_len)

        pltpu.sync_copy(agg_v, out_hbm.at[wid])

    return msg_kernel(edges2, y_flat)


def _final_call(agg_part, y, dinv, b, W_lin, b_lin, n, p_stride, d_hid,
                d_out):
    def body(ap_ref, y_ref, dinv_ref, b_ref, wl_ref, bl_ref, h_ref, z_ref):
        accr = jnp.sum(ap_ref[...], axis=0, keepdims=True)
        aggs = jnp.concatenate(
            [accr[:, c * p_stride:c * p_stride + n]
             for c in range(d_hid)], axis=0)
        dinv = dinv_ref[...]
        aggc = (aggs + y_ref[...]) * dinv
        b_col = b_ref[...].reshape(d_hid, 1)
        h_cols = jnp.maximum(aggc + b_col, 0.0)
        z_cols = lax.dot_general(
            wl_ref[...], h_cols, (((0,), (0,)), ((), ())),
            preferred_element_type=jnp.float32)
        z_cols = z_cols + bl_ref[...].reshape(d_out, 1)
        h_ref[...] = h_cols
        z_ref[...] = z_cols

    return pl.pallas_call(
        body,
        out_shape=[
            jax.ShapeDtypeStruct((d_hid, n), jnp.float32),
            jax.ShapeDtypeStruct((d_out, n), jnp.float32),
        ],
    )(agg_part, y, dinv, b, W_lin, b_lin)


def kernel(x, edges, W, b, W_lin, b_lin):
    n, d_in = x.shape
    d_hid = W.shape[1]
    d_out = W_lin.shape[1]
    e = edges.shape[1]
    assert e % (_NW * _L) == 0 and n % _L == 0

    edges = edges.astype(jnp.int32)

    xwT = _xwT_call(x, W, n, d_in, d_hid)
    deg_part = _sc_degree_call(edges, n, e)
    y, y_flat, dinv = _dinv_y_call(deg_part, xwT, n, d_hid)
    agg_part = _sc_msgpass_call(edges, y_flat, n, e, d_hid)
    p_stride = agg_part.shape[1] // d_hid
    h_cols, z_cols = _final_call(agg_part, y, dinv, b, W_lin,
                                 b_lin, n, p_stride, d_hid, d_out)
    return (h_cols.T, z_cols.T)
